# Initial kernel scaffold; baseline (speedup 1.0000x reference)
import jax
import jax.numpy as jnp
from jax.experimental import pallas as pl


def kernel(labels, E):
    raise NotImplementedError("write your pallas kernel here")


# speedup vs baseline: 1.3071x; 1.3071x over previous
import functools

import jax
import jax.numpy as jnp
from jax import lax
from jax.experimental import pallas as pl
from jax.experimental.pallas import tpu as pltpu
from jax.experimental.pallas import tpu_sc as plsc

DIM = 32
GROUP = 128
NC = 2
NS = 16
NW = NC * NS


@functools.lru_cache(maxsize=None)
def _build(batch, hist, vocab):
    total = batch * hist
    assert total % (NW * GROUP) == 0
    groups_per_w = total // (NW * GROUP)
    mesh = plsc.VectorSubcoreMesh(core_axis_name="c", subcore_axis_name="s")

    @functools.partial(
        pl.kernel,
        mesh=mesh,
        out_type=jax.ShapeDtypeStruct((total, DIM), jnp.float32),
        compiler_params=pltpu.CompilerParams(use_tc_tiling_on_sc=False),
        scratch_types=[
            pltpu.VMEM((groups_per_w, GROUP), jnp.int32),
            pltpu.VMEM((GROUP, DIM), jnp.float32),
            pltpu.SemaphoreType.DMA,
        ],
    )
    def k(idx_hbm, table_hbm, out_hbm, idx_v, rows_v, sem):
        wid = lax.axis_index("s") * NC + lax.axis_index("c")
        base = wid * (groups_per_w * GROUP)
        pltpu.sync_copy(idx_hbm.at[wid], idx_v)

        def body(g, carry):
            pltpu.async_copy(table_hbm.at[idx_v.at[g]], rows_v, sem).wait()
            pltpu.sync_copy(rows_v, out_hbm.at[pl.ds(base + g * GROUP, GROUP)])
            return carry

        lax.fori_loop(0, groups_per_w, body, 0)

    return k


def kernel(labels, E):
    batch, hist = labels.shape
    vocab, dim = E.shape
    assert dim == DIM
    total = batch * hist
    idx = labels.astype(jnp.int32).reshape(NW, total // (NW * GROUP), GROUP)
    out = _build(batch, hist, vocab)(idx, E)
    return out.reshape(batch, hist, DIM)


# speedup vs baseline: 1.5014x; 1.1487x over previous
import functools

import jax
import jax.numpy as jnp
from jax import lax
from jax.experimental import pallas as pl
from jax.experimental.pallas import tpu as pltpu
from jax.experimental.pallas import tpu_sc as plsc

DIM = 32
GROUP = 128
K = 10
NC = 2
NS = 16
NW = NC * NS
CHUNK = K * GROUP


@functools.lru_cache(maxsize=None)
def _build(total):
    assert total % (NW * CHUNK) == 0
    chunks_per_w = total // (NW * CHUNK)
    groups_per_w = chunks_per_w * K
    assert chunks_per_w % 2 == 0
    mesh = plsc.VectorSubcoreMesh(core_axis_name="c", subcore_axis_name="s")

    @functools.partial(
        pl.kernel,
        mesh=mesh,
        out_type=jax.ShapeDtypeStruct((total, DIM), jnp.float32),
        compiler_params=pltpu.CompilerParams(use_tc_tiling_on_sc=False),
        scratch_types=[
            pltpu.VMEM((groups_per_w, GROUP), jnp.int32),
            pltpu.VMEM((2, CHUNK, DIM), jnp.float32),
            pltpu.SemaphoreType.DMA((2,)),
            pltpu.SemaphoreType.DMA((2,)),
        ],
    )
    def k(idx_hbm, table_hbm, out_hbm, idx_v, rows_v, gsem, wsem):
        wid = lax.axis_index("s") * NC + lax.axis_index("c")
        base = wid * (chunks_per_w * CHUNK)
        pltpu.sync_copy(idx_hbm.at[wid], idx_v)

        def fire_chunk(c, p):
            for j in range(K):
                pltpu.async_copy(
                    table_hbm.at[idx_v.at[c * K + j]],
                    rows_v.at[p].at[pl.ds(j * GROUP, GROUP)],
                    gsem.at[p],
                )

        def drain_gathers(p):
            pltpu.make_async_copy(
                out_hbm.at[pl.ds(0, CHUNK)], rows_v.at[p], gsem.at[p]
            ).wait()

        def drain_write(p):
            pltpu.make_async_copy(
                rows_v.at[p], out_hbm.at[pl.ds(0, CHUNK)], wsem.at[p]
            ).wait()

        fire_chunk(0, 0)
        fire_chunk(1, 1)

        def body(i, carry):
            for p in range(2):
                c = 2 * i + p
                drain_gathers(p)
                pltpu.make_async_copy(
                    rows_v.at[p],
                    out_hbm.at[pl.ds(base + c * CHUNK, CHUNK)],
                    wsem.at[p],
                ).start()

                @pl.when(i < chunks_per_w // 2 - 1)
                def _():
                    drain_write(p)
                    fire_chunk(c + 2, p)

            return carry

        lax.fori_loop(0, chunks_per_w // 2, body, 0)
        drain_write(0)
        drain_write(1)

    return k


def kernel(labels, E):
    batch, hist = labels.shape
    vocab, dim = E.shape
    assert dim == DIM
    total = batch * hist
    idx = labels.astype(jnp.int32).reshape(NW, total // (NW * GROUP), GROUP)
    out = _build(total)(idx, E)
    return out.reshape(batch, hist, DIM)
